# baseline (speedup 1.0000x reference)
import functools

import jax
import jax.numpy as jnp
from jax import lax
from jax.experimental import pallas as pl
from jax.experimental.pallas import tpu as pltpu
from jax.experimental.pallas import tpu_sc as plsc

_CHUNK = 128


def _make_gather(num_rows, dim, batch_size):
    info = plsc.get_sparse_core_info()
    nc, ns = info.num_cores, info.num_subcores
    nw = nc * ns
    assert batch_size % (nw * _CHUNK) == 0
    b_per_w = batch_size // nw
    n_chunks = b_per_w // _CHUNK

    mesh = plsc.VectorSubcoreMesh(core_axis_name="c", subcore_axis_name="s")

    @functools.partial(
        pl.kernel,
        mesh=mesh,
        out_type=jax.ShapeDtypeStruct((batch_size, dim), jnp.float32),
        scratch_types=[
            pltpu.VMEM((n_chunks, _CHUNK), jnp.int32),
            pltpu.VMEM((b_per_w, dim), jnp.float32),
            pltpu.SemaphoreType.DMA,
        ],
        compiler_params=pltpu.CompilerParams(use_tc_tiling_on_sc=False),
    )
    def gather_kernel(table_hbm, idx_hbm, out_hbm, idx_v, rows_v, sem):
        wid = lax.axis_index("s") * nc + lax.axis_index("c")
        base = wid * b_per_w
        pltpu.sync_copy(idx_hbm.at[wid], idx_v)
        copies = []
        for j in range(n_chunks):
            copies.append(
                pltpu.make_async_copy(
                    table_hbm.at[idx_v.at[j]],
                    rows_v.at[pl.ds(j * _CHUNK, _CHUNK)],
                    sem,
                )
            )
        for c in copies:
            c.start()
        for c in copies:
            c.wait()
        pltpu.sync_copy(rows_v, out_hbm.at[pl.ds(base, b_per_w)])

    return gather_kernel, nw, n_chunks


def kernel(batch, table):
    batch_size = batch.shape[0]
    num_rows, dim = table.shape
    gather_fn, nw, n_chunks = _make_gather(num_rows, dim, batch_size)
    idx = batch.reshape(nw, n_chunks, _CHUNK)
    return gather_fn(table, idx)


# speedup vs baseline: 1.7297x; 1.7297x over previous
import functools

import jax
import jax.numpy as jnp
from jax import lax
from jax.experimental import pallas as pl
from jax.experimental.pallas import tpu as pltpu
from jax.experimental.pallas import tpu_sc as plsc

_LANES = 16


def _make_gather(num_rows, dim, batch_size):
    info = plsc.get_sparse_core_info()
    nc, ns = info.num_cores, info.num_subcores
    nw = nc * ns
    assert batch_size % (nw * _LANES) == 0
    b_per_w = batch_size // nw
    n_vecs = b_per_w // _LANES

    mesh = plsc.VectorSubcoreMesh(core_axis_name="c", subcore_axis_name="s")

    @functools.partial(
        pl.kernel,
        mesh=mesh,
        out_type=jax.ShapeDtypeStruct((batch_size, dim), jnp.float32),
        scratch_types=[
            pltpu.VMEM((b_per_w,), jnp.int32),
            pltpu.VMEM((b_per_w, dim), jnp.float32),
            pltpu.SemaphoreType.DMA,
        ],
    )
    def gather_kernel(table_hbm, idx_hbm, out_hbm, idx_v, rows_v, sem):
        wid = lax.axis_index("s") * nc + lax.axis_index("c")
        base = wid * b_per_w
        pltpu.sync_copy(idx_hbm.at[pl.ds(base, b_per_w)], idx_v)

        def issue_vec(k, carry):
            iv = idx_v[pl.ds(k * _LANES, _LANES)]
            for j in range(_LANES):
                r = lax.squeeze(lax.slice(iv, (j,), (j + 1,)), (0,))
                pltpu.make_async_copy(
                    table_hbm.at[pl.ds(r, 1)],
                    rows_v.at[pl.ds(k * _LANES + j, 1)],
                    sem,
                ).start()
            return carry

        lax.fori_loop(0, n_vecs, issue_vec, 0)

        def drain(k, carry):
            pltpu.make_async_copy(
                table_hbm.at[pl.ds(0, 1)],
                rows_v.at[pl.ds(0, 1)],
                sem,
            ).wait()
            return carry

        lax.fori_loop(0, b_per_w, drain, 0)
        pltpu.sync_copy(rows_v, out_hbm.at[pl.ds(base, b_per_w)])

    return gather_kernel, nw


def kernel(batch, table):
    batch_size = batch.shape[0]
    num_rows, dim = table.shape
    gather_fn, nw = _make_gather(num_rows, dim, batch_size)
    return gather_fn(table, batch)


# speedup vs baseline: 1.8059x; 1.0441x over previous
import functools

import jax
import jax.numpy as jnp
from jax import lax
from jax.experimental import pallas as pl
from jax.experimental.pallas import tpu as pltpu
from jax.experimental.pallas import tpu_sc as plsc

_L = 16
_CHUNK_LT = 4
_CHUNK = _CHUNK_LT * 128
_RING = 128


def _lane(v, j):
    return lax.squeeze(lax.slice(v, (j,), (j + 1,)), (0,))


def _make_gather(num_rows, dim, batch_size):
    info = plsc.get_sparse_core_info()
    nc, ns = info.num_cores, info.num_subcores
    nw = nc * ns
    full_lt = num_rows // 128
    tail_start = full_lt * 128
    tail_n = num_rows - tail_start
    assert tail_n > 0
    w_rows = -(-num_rows // (nw * 128)) * 128
    n_chunks = -(-(w_rows // 128) // _CHUNK_LT)
    clamp_lt = full_lt - _CHUNK_LT
    n_vec_all = batch_size // _L

    mesh = plsc.VectorSubcoreMesh(core_axis_name="c", subcore_axis_name="s")

    @functools.partial(
        pl.kernel,
        mesh=mesh,
        out_type=jax.ShapeDtypeStruct((batch_size, dim), jnp.float32),
        scratch_types=[
            pltpu.VMEM((batch_size,), jnp.int32),
            pltpu.VMEM((batch_size + _L,), jnp.int32),
            pltpu.VMEM((batch_size + _L,), jnp.int32),
            pltpu.VMEM((dim, _CHUNK), jnp.float32),
            pltpu.VMEM((dim, tail_n), jnp.float32),
            pltpu.VMEM((_RING, dim), jnp.float32),
            pltpu.SemaphoreType.DMA,
        ],
        compiler_params=pltpu.CompilerParams(needs_layout_passes=False),
    )
    def gather_kernel(
        tableT_hbm, idx_hbm, tail_hbm, out_hbm,
        idx_all, my_r, my_i, buf, tailbuf, ring, sem,
    ):
        wid = lax.axis_index("s") * nc + lax.axis_index("c")
        lo = wid * w_rows
        hi = lo + w_rows
        pltpu.sync_copy(idx_hbm, idx_all)

        def scan_v(t, n):
            iv = idx_all[pl.ds(t * _L, _L)]
            pos = lax.iota(jnp.int32, _L) + t * _L
            m = (iv >= lo) & (iv < hi)
            cnt = _lane(plsc.all_reduce_population_count(m), 0)
            plsc.store_compressed(my_r.at[pl.ds(n, _L)], iv, mask=m)
            plsc.store_compressed(my_i.at[pl.ds(n, _L)], pos, mask=m)
            return n + cnt

        n = lax.fori_loop(0, n_vec_all, scan_v, 0)
        my_r[pl.ds(n, _L)] = jnp.full((_L,), jnp.int32(1 << 30))
        nv = (n + _L - 1) // _L

        def drain_full():
            pltpu.make_async_copy(
                out_hbm.at[pl.ds(0, _RING)], ring, sem
            ).wait()
            return jnp.int32(0)

        def process(src_ref, clo, chi, sub_base, nb):
            def vec_body(t, nb):
                mv = my_r[pl.ds(t * _L, _L)]
                pv = my_i[pl.ds(t * _L, _L)]
                m = (mv >= clo) & (mv < chi)

                def do(nb):
                    mi = m.astype(jnp.int32)
                    for j in range(_L):
                        valid = _lane(mi, j) == 1
                        rl = _lane(mv, j) - sub_base
                        pos = _lane(pv, j)

                        def issue(nb, rl=rl, pos=pos):
                            slot = nb
                            for k in range(dim // _L):
                                g = plsc.load_gather(
                                    src_ref,
                                    [
                                        lax.iota(jnp.int32, _L) + k * _L,
                                        jnp.full((_L,), rl),
                                    ],
                                )
                                ring[slot, pl.ds(k * _L, _L)] = g
                            pltpu.make_async_copy(
                                ring.at[pl.ds(slot, 1)],
                                out_hbm.at[pl.ds(pos, 1)],
                                sem,
                            ).start()
                            nb2 = nb + 1
                            return lax.cond(
                                nb2 >= _RING, drain_full, lambda: nb2
                            )

                        nb = lax.cond(valid, issue, lambda nb: nb, nb)
                    return nb

                return lax.cond(jnp.any(m), do, lambda nb: nb, nb)

            return lax.fori_loop(0, nv, vec_body, nb)

        def chunk_body(c, nb):
            s_lt = jnp.minimum(wid * (w_rows // 128) + c * _CHUNK_LT, clamp_lt)
            off = pl.multiple_of(s_lt * 128, 128)
            pltpu.sync_copy(tableT_hbm.at[:, pl.ds(off, _CHUNK)], buf)
            return process(buf, off, off + _CHUNK, off, nb)

        nb = lax.fori_loop(0, n_chunks, chunk_body, 0)

        pltpu.sync_copy(tail_hbm, tailbuf)
        nb = process(tailbuf, tail_start, num_rows, tail_start, nb)

        def drain1(k, c):
            pltpu.make_async_copy(
                out_hbm.at[pl.ds(0, 1)], ring.at[pl.ds(0, 1)], sem
            ).wait()
            return c

        lax.fori_loop(0, nb, drain1, 0)

    return gather_kernel, tail_start


def kernel(batch, table):
    batch_size = batch.shape[0]
    num_rows, dim = table.shape
    gather_fn, tail_start = _make_gather(num_rows, dim, batch_size)
    return gather_fn(table.T, batch, table[tail_start:, :].T)


# speedup vs baseline: 2.9562x; 1.6369x over previous
import functools

import jax
import jax.numpy as jnp
from jax import lax
from jax.experimental import pallas as pl
from jax.experimental.pallas import tpu as pltpu
from jax.experimental.pallas import tpu_sc as plsc

_L = 16
_CHUNK_LT = 2
_CHUNK = _CHUNK_LT * 128
_RING = 128


def _lane(v, j):
    return lax.squeeze(lax.slice(v, (j,), (j + 1,)), (0,))


def _sload(ref, q):
    return _lane(ref[pl.ds(q, _L)], 0)


def _make_gather(num_rows, dim, batch_size):
    info = plsc.get_sparse_core_info()
    nc, ns = info.num_cores, info.num_subcores
    nw = nc * ns
    full_lt = num_rows // 128
    tail_start = full_lt * 128
    tail_n = num_rows - tail_start
    assert tail_n > 0
    w_rows = -(-num_rows // (nw * 128)) * 128
    w_lt = w_rows // 128
    n_chunks = -(-w_lt // _CHUNK_LT)
    n_chunks += n_chunks % 2
    clamp_lt = full_lt - _CHUNK_LT
    n_vec_all = batch_size // _L
    tail_bin = n_chunks

    mesh = plsc.VectorSubcoreMesh(core_axis_name="c", subcore_axis_name="s")

    @functools.partial(
        pl.kernel,
        mesh=mesh,
        out_type=jax.ShapeDtypeStruct((batch_size, dim), jnp.float32),
        scratch_types=[
            pltpu.VMEM((batch_size + _L,), jnp.int32),
            pltpu.VMEM((batch_size + _L,), jnp.int32),
            pltpu.VMEM((batch_size + _L,), jnp.int32),
            pltpu.VMEM((n_chunks + _L,), jnp.int32),
            pltpu.VMEM((dim, _CHUNK), jnp.float32),
            pltpu.VMEM((dim, _CHUNK), jnp.float32),
            pltpu.VMEM((dim, tail_n), jnp.float32),
            pltpu.VMEM((_RING, dim), jnp.float32),
            pltpu.SemaphoreType.DMA,
            pltpu.SemaphoreType.DMA,
            pltpu.SemaphoreType.DMA,
        ],
        compiler_params=pltpu.CompilerParams(needs_layout_passes=False),
    )
    def gather_kernel(
        tableT_hbm, idx_hbm, tail_hbm, out_hbm,
        idx_all, my_i, my_i2, cur, buf0, buf1, tailbuf, ring,
        sem0, sem1, semo,
    ):
        wid = lax.axis_index("s") * nc + lax.axis_index("c")
        lo = wid * w_rows
        hi = lo + w_rows
        pltpu.sync_copy(idx_hbm, idx_all.at[pl.ds(0, batch_size)])
        m0 = lax.iota(jnp.int32, _L) == 0

        def _sstore(ref, b, val):
            plsc.store_scatter(
                ref,
                [jnp.full((_L,), b, jnp.int32)],
                jnp.full((_L,), val, jnp.int32),
                mask=m0,
            )
        pltpu.sync_copy(tail_hbm, tailbuf)

        def scan_v(t, n):
            iv = idx_all[pl.ds(t * _L, _L)]
            pos = lax.iota(jnp.int32, _L) + t * _L
            m = (iv >= lo) & (iv < hi)
            cnt = _lane(plsc.all_reduce_population_count(m), 0)
            plsc.store_compressed(my_i.at[pl.ds(n, _L)], pos, mask=m)
            return n + cnt

        n = lax.fori_loop(0, n_vec_all, scan_v, 0)

        def item_bin(r):
            return jnp.where(
                r >= tail_start, tail_bin, (r - lo) >> 8
            ).astype(jnp.int32)

        for t in range((n_chunks + _L) // _L):
            cur[pl.ds(t * _L, _L)] = jnp.zeros((_L,), jnp.int32)

        def hist(q, c):
            b = item_bin(_sload(idx_all, _sload(my_i, q)))
            _sstore(cur, b, _sload(cur, b) + 1)
            return c

        lax.fori_loop(0, n, hist, 0)

        def prefix(b, s):
            c = _sload(cur, b)
            _sstore(cur, b, s)
            return s + c

        lax.fori_loop(0, tail_bin + 1, prefix, 0)

        def scatter(q, c):
            i = _sload(my_i, q)
            b = item_bin(_sload(idx_all, i))
            p = _sload(cur, b)
            _sstore(my_i2, p, i)
            _sstore(cur, b, p + 1)
            return c

        lax.fori_loop(0, n, scatter, 0)

        def drain_full():
            pltpu.make_async_copy(
                out_hbm.at[pl.ds(0, _RING)], ring, sem=semo
            ).wait()
            return jnp.int32(0)

        def serve(q, nb, src_ref, sub_base):
            i = _sload(my_i2, q)
            rl = _sload(idx_all, i) - sub_base
            slot = nb
            for k in range(dim // _L):
                g = plsc.load_gather(
                    src_ref,
                    [lax.iota(jnp.int32, _L) + k * _L, jnp.full((_L,), rl)],
                )
                ring[slot, pl.ds(k * _L, _L)] = g
            pltpu.make_async_copy(
                ring.at[pl.ds(slot, 1)], out_hbm.at[pl.ds(i, 1)], semo
            ).start()
            nb2 = nb + 1
            return lax.cond(nb2 >= _RING, drain_full, lambda: nb2)

        def chunk_off(c):
            s_lt = jnp.minimum(wid * w_lt + c * _CHUNK_LT, clamp_lt)
            return pl.multiple_of(s_lt * 128, 128)

        def start_chunk(c, buf, sem):
            pltpu.make_async_copy(
                tableT_hbm.at[:, pl.ds(chunk_off(c), _CHUNK)], buf, sem
            ).start()

        def wait_chunk(c, buf, sem):
            pltpu.make_async_copy(
                tableT_hbm.at[:, pl.ds(chunk_off(c), _CHUNK)], buf, sem
            ).wait()

        def bin_range(c):
            qlo = jnp.where(c > 0, _sload(cur, jnp.maximum(c - 1, 0)), 0)
            return qlo, _sload(cur, c)

        def process_chunk(c, buf, nb):
            base = chunk_off(c)
            qlo, qhi = bin_range(c)
            return lax.fori_loop(
                qlo, qhi, lambda q, nb: serve(q, nb, buf, base), nb
            )

        start_chunk(0, buf0, sem0)

        def pair_body(p, nb):
            c0 = p * 2
            start_chunk(c0 + 1, buf1, sem1)
            wait_chunk(c0, buf0, sem0)
            nb = process_chunk(c0, buf0, nb)
            start_chunk(c0 + 2, buf0, sem0)
            wait_chunk(c0 + 1, buf1, sem1)
            return process_chunk(c0 + 1, buf1, nb)

        nb = lax.fori_loop(0, n_chunks // 2, pair_body, 0)
        wait_chunk(n_chunks, buf0, sem0)

        qlo, qhi = bin_range(tail_bin)
        nb = lax.fori_loop(
            qlo, qhi, lambda q, nb: serve(q, nb, tailbuf, tail_start), nb
        )

        def drain1(k, c):
            pltpu.make_async_copy(
                out_hbm.at[pl.ds(0, 1)], ring.at[pl.ds(0, 1)], semo
            ).wait()
            return c

        lax.fori_loop(0, nb, drain1, 0)

    return gather_kernel, tail_start


def kernel(batch, table):
    batch_size = batch.shape[0]
    num_rows, dim = table.shape
    gather_fn, tail_start = _make_gather(num_rows, dim, batch_size)
    return gather_fn(table.T, batch, table[tail_start:, :].T)


# speedup vs baseline: 3.1983x; 1.0819x over previous
import functools

import jax
import jax.numpy as jnp
from jax import lax
from jax.experimental import pallas as pl
from jax.experimental.pallas import tpu as pltpu
from jax.experimental.pallas import tpu_sc as plsc

_L = 16
_CHUNK_LT = 4
_CHUNK = _CHUNK_LT * 128
_RING = 64


def _lane(v, j):
    return lax.squeeze(lax.slice(v, (j,), (j + 1,)), (0,))


def _sload(ref, q):
    return _lane(ref[pl.ds(q, _L)], 0)


def _make_gather(num_rows, dim, batch_size):
    info = plsc.get_sparse_core_info()
    nc, ns = info.num_cores, info.num_subcores
    nw = nc * ns
    full_lt = num_rows // 128
    tail_start = full_lt * 128
    tail_n = num_rows - tail_start
    assert tail_n > 0
    w_rows = -(-num_rows // (nw * 128)) * 128
    w_lt = w_rows // 128
    n_chunks = -(-w_lt // _CHUNK_LT)
    n_chunks += n_chunks % 2
    clamp_lt = full_lt - _CHUNK_LT
    n_vec_all = batch_size // _L
    tail_bin = n_chunks

    mesh = plsc.VectorSubcoreMesh(core_axis_name="c", subcore_axis_name="s")

    @functools.partial(
        pl.kernel,
        mesh=mesh,
        out_type=jax.ShapeDtypeStruct((batch_size, dim), jnp.float32),
        scratch_types=[
            pltpu.VMEM((batch_size + _L,), jnp.int32),
            pltpu.VMEM((batch_size + _L,), jnp.int32),
            pltpu.VMEM((batch_size + _L,), jnp.int32),
            pltpu.VMEM((n_chunks + _L,), jnp.int32),
            pltpu.VMEM((dim, _CHUNK), jnp.float32),
            pltpu.VMEM((dim, _CHUNK), jnp.float32),
            pltpu.VMEM((dim * tail_n,), jnp.float32),
            pltpu.VMEM((_RING, dim), jnp.float32),
            pltpu.SemaphoreType.DMA,
            pltpu.SemaphoreType.DMA,
            pltpu.SemaphoreType.DMA,
        ],
        compiler_params=pltpu.CompilerParams(needs_layout_passes=False),
    )
    def gather_kernel(
        tableT_hbm, idx_hbm, tail_hbm, out_hbm,
        idx_all, my_i, my_i2, cur, buf0, buf1, tailbuf, ring,
        sem0, sem1, semo,
    ):
        wid = lax.axis_index("s") * nc + lax.axis_index("c")
        lo = wid * w_rows
        hi = lo + w_rows
        pltpu.sync_copy(idx_hbm, idx_all.at[pl.ds(0, batch_size)])
        m0 = lax.iota(jnp.int32, _L) == 0

        def _sstore(ref, b, val):
            plsc.store_scatter(
                ref,
                [jnp.full((_L,), b, jnp.int32)],
                jnp.full((_L,), val, jnp.int32),
                mask=m0,
            )
        pltpu.sync_copy(tail_hbm, tailbuf)

        def scan_v(t, n):
            iv = idx_all[pl.ds(t * _L, _L)]
            pos = lax.iota(jnp.int32, _L) + t * _L
            m = (iv >= lo) & (iv < hi)
            cnt = _lane(plsc.all_reduce_population_count(m), 0)
            plsc.store_compressed(my_i.at[pl.ds(n, _L)], pos, mask=m)
            return n + cnt

        n = lax.fori_loop(0, n_vec_all, scan_v, 0)

        def item_bin(r):
            return jnp.where(
                r >= tail_start, tail_bin, (r - lo) >> 9
            ).astype(jnp.int32)

        for t in range((n_chunks + _L) // _L):
            cur[pl.ds(t * _L, _L)] = jnp.zeros((_L,), jnp.int32)

        def hist(q, c):
            b = item_bin(_sload(idx_all, _sload(my_i, q)))
            _sstore(cur, b, _sload(cur, b) + 1)
            return c

        lax.fori_loop(0, n, hist, 0)

        def prefix(b, s):
            c = _sload(cur, b)
            _sstore(cur, b, s)
            return s + c

        lax.fori_loop(0, tail_bin + 1, prefix, 0)

        def scatter(q, c):
            i = _sload(my_i, q)
            b = item_bin(_sload(idx_all, i))
            p = _sload(cur, b)
            _sstore(my_i2, p, i)
            _sstore(cur, b, p + 1)
            return c

        lax.fori_loop(0, n, scatter, 0)

        def drain_full():
            pltpu.make_async_copy(
                out_hbm.at[pl.ds(0, _RING)], ring, sem=semo
            ).wait()
            return jnp.int32(0)

        def serve(q, nb, src_ref, sub_base, row_w=None):
            i = _sload(my_i2, q)
            rl = _sload(idx_all, i) - sub_base
            slot = nb
            for k in range(dim // _L):
                dims = lax.iota(jnp.int32, _L) + k * _L
                if row_w is None:
                    g = plsc.load_gather(
                        src_ref, [dims, jnp.full((_L,), rl)]
                    )
                else:
                    g = plsc.load_gather(
                        src_ref, [dims * row_w + rl]
                    )
                ring[slot, pl.ds(k * _L, _L)] = g
            pltpu.make_async_copy(
                ring.at[pl.ds(slot, 1)], out_hbm.at[pl.ds(i, 1)], semo
            ).start()
            nb2 = nb + 1
            return lax.cond(nb2 >= _RING, drain_full, lambda: nb2)

        def chunk_off(c):
            s_lt = jnp.minimum(wid * w_lt + c * _CHUNK_LT, clamp_lt)
            return pl.multiple_of(s_lt * 128, 128)

        def start_chunk(c, buf, sem):
            pltpu.make_async_copy(
                tableT_hbm.at[:, pl.ds(chunk_off(c), _CHUNK)], buf, sem
            ).start()

        def wait_chunk(c, buf, sem):
            pltpu.make_async_copy(
                tableT_hbm.at[:, pl.ds(chunk_off(c), _CHUNK)], buf, sem
            ).wait()

        def bin_range(c):
            qlo = jnp.where(c > 0, _sload(cur, jnp.maximum(c - 1, 0)), 0)
            return qlo, _sload(cur, c)

        def process_chunk(c, buf, nb):
            base = chunk_off(c)
            qlo, qhi = bin_range(c)
            return lax.fori_loop(
                qlo, qhi, lambda q, nb: serve(q, nb, buf, base), nb
            )

        start_chunk(0, buf0, sem0)

        def pair_body(p, nb):
            c0 = p * 2
            start_chunk(c0 + 1, buf1, sem1)
            wait_chunk(c0, buf0, sem0)
            nb = process_chunk(c0, buf0, nb)
            start_chunk(c0 + 2, buf0, sem0)
            wait_chunk(c0 + 1, buf1, sem1)
            return process_chunk(c0 + 1, buf1, nb)

        nb = lax.fori_loop(0, n_chunks // 2, pair_body, 0)
        wait_chunk(n_chunks, buf0, sem0)

        qlo, qhi = bin_range(tail_bin)
        nb = lax.fori_loop(
            qlo, qhi,
            lambda q, nb: serve(q, nb, tailbuf, tail_start, row_w=tail_n),
            nb,
        )

        def drain1(k, c):
            pltpu.make_async_copy(
                out_hbm.at[pl.ds(0, 1)], ring.at[pl.ds(0, 1)], semo
            ).wait()
            return c

        lax.fori_loop(0, nb, drain1, 0)

    return gather_kernel, tail_start


def kernel(batch, table):
    batch_size = batch.shape[0]
    num_rows, dim = table.shape
    gather_fn, tail_start = _make_gather(num_rows, dim, batch_size)
    return gather_fn(table.T, batch, table[tail_start:, :].T.reshape(-1))
